# baseline (speedup 1.0000x reference)
import functools

import jax
import jax.numpy as jnp
from jax import lax
from jax.experimental import pallas as pl
from jax.experimental.pallas import tpu as pltpu
from jax.experimental.pallas import tpu_sc as plsc

_NUM_CODONS = 64
_EMBED_DIM = 16

_B = 16384
_T = 200
_N = _B * _T

_INFO = plsc.get_sparse_core_info()
_NC = _INFO.num_cores
_NS = _INFO.num_subcores
_NW = _NC * _NS
_PER_W = _N // _NW
_CHUNK = 3200
_STEPS = _PER_W // _CHUNK
_GRP = 16


def _gather_kernel(x_hbm, table_hbm, out_hbm,
                   table_v, idx_v, rows0_v, rows1_v, sem0, sem1):
    wid = lax.axis_index("s") * _NC + lax.axis_index("c")
    base = wid * _PER_W
    pltpu.sync_copy(table_hbm, table_v)
    rows_bufs = (rows0_v, rows1_v)
    sems = (sem0, sem1)

    lanes = lax.iota(jnp.int32, _GRP) * _EMBED_DIM

    def compute(rows_v):
        def grp(j, c):
            tok = idx_v[pl.ds(j * _GRP, _GRP)]
            g = tok * _EMBED_DIM
            sbase = lanes + j * (_GRP * _EMBED_DIM)
            for d in range(_EMBED_DIM):
                vals = plsc.load_gather(table_v, [g + d])
                plsc.store_scatter(rows_v, [sbase + d], vals)
            return c
        lax.fori_loop(0, _CHUNK // _GRP, grp, 0, unroll=False)

    def body(i2, carry):
        for b in range(2):
            i = i2 * 2 + b
            off = base + i * _CHUNK
            out_slice = out_hbm.at[pl.ds(off * _EMBED_DIM, _CHUNK * _EMBED_DIM)]
            pltpu.sync_copy(x_hbm.at[pl.ds(off, _CHUNK)], idx_v)

            @pl.when(i2 > 0)
            def _drain():
                pltpu.make_async_copy(rows_bufs[b], out_slice, sems[b]).wait()

            compute(rows_bufs[b])
            pltpu.async_copy(rows_bufs[b], out_slice, sems[b])
        return carry

    lax.fori_loop(0, _STEPS // 2, body, 0)
    tail0 = base + (_STEPS - 2) * _CHUNK
    tail1 = base + (_STEPS - 1) * _CHUNK
    pltpu.make_async_copy(
        rows0_v, out_hbm.at[pl.ds(tail0 * _EMBED_DIM, _CHUNK * _EMBED_DIM)],
        sem0).wait()
    pltpu.make_async_copy(
        rows1_v, out_hbm.at[pl.ds(tail1 * _EMBED_DIM, _CHUNK * _EMBED_DIM)],
        sem1).wait()


@jax.jit
def _run(x_flat, table_flat):
    mesh = plsc.VectorSubcoreMesh(core_axis_name="c", subcore_axis_name="s")
    kern = functools.partial(
        pl.kernel,
        mesh=mesh,
        out_type=jax.ShapeDtypeStruct((_N * _EMBED_DIM,), jnp.float32),
        scratch_types=[
            pltpu.VMEM((_NUM_CODONS * _EMBED_DIM,), jnp.float32),
            pltpu.VMEM((_CHUNK,), jnp.int32),
            pltpu.VMEM((_CHUNK * _EMBED_DIM,), jnp.float32),
            pltpu.VMEM((_CHUNK * _EMBED_DIM,), jnp.float32),
            pltpu.SemaphoreType.DMA,
            pltpu.SemaphoreType.DMA,
        ],
        compiler_params=pltpu.CompilerParams(use_tc_tiling_on_sc=True, needs_layout_passes=False),
    )(_gather_kernel)
    return kern(x_flat, table_flat)


def kernel(x, embeddings):
    out = _run(x.reshape(_N), embeddings.reshape(_NUM_CODONS * _EMBED_DIM))
    return out.reshape(_B, _T, _EMBED_DIM)


# speedup vs baseline: 8.2875x; 8.2875x over previous
import functools

import jax
import jax.numpy as jnp
from jax import lax
from jax.experimental import pallas as pl
from jax.experimental.pallas import tpu as pltpu
from jax.experimental.pallas import tpu_sc as plsc

_NUM_CODONS = 64
_EMBED_DIM = 16

_B = 16384
_T = 200
_N = _B * _T

_INFO = plsc.get_sparse_core_info()
_NC = _INFO.num_cores
_NS = _INFO.num_subcores
_NW = _NC * _NS
_BW = _B // _NW
_GRP = 16
_NG = _BW // _GRP


def _gather_kernel(xt_hbm, tableT_hbm, out_hbm, table_v, idx_v,
                   buf0_v, buf1_v, sem0, sem1):
    wid = lax.axis_index("s") * _NC + lax.axis_index("c")
    b0 = wid * _BW
    btile0 = wid * (_BW // 128)
    pltpu.sync_copy(tableT_hbm, table_v)
    bufs = (buf0_v, buf1_v)
    sems = (sem0, sem1)

    def compute(buf_v, dband):
        for bg in range(_NG):
            tok = idx_v[pl.ds(bg * _GRP, _GRP)]
            vals = [plsc.load_gather(table_v, [tok + (dband * 8 + ds) * _NUM_CODONS])
                    for ds in range(8)]
            for ds in range(8):
                buf_v[bg // 8, ds, pl.ds((bg % 8) * _GRP, _GRP)] = vals[ds]

    def body(t, carry):
        pltpu.sync_copy(xt_hbm.at[pl.ds(t * _B + b0, _BW)], idx_v)
        for dband in range(2):
            out_slice = out_hbm.at[t, dband, pl.ds(btile0, _BW // 128)]

            @pl.when(t > 0)
            def _drain():
                pltpu.make_async_copy(bufs[dband], out_slice, sems[dband]).wait()

            compute(bufs[dband], dband)
            pltpu.async_copy(bufs[dband], out_slice, sems[dband])
        return carry

    lax.fori_loop(0, _T, body, 0)
    for dband in range(2):
        pltpu.make_async_copy(
            bufs[dband],
            out_hbm.at[_T - 1, dband, pl.ds(btile0, _BW // 128)],
            sems[dband]).wait()


@jax.jit
def _run(xt_flat, tableT_flat):
    mesh = plsc.VectorSubcoreMesh(core_axis_name="c", subcore_axis_name="s")
    kern = functools.partial(
        pl.kernel,
        mesh=mesh,
        out_type=jax.ShapeDtypeStruct((_T, 2, _B // 128, 8, 128), jnp.float32),
        scratch_types=[
            pltpu.VMEM((_NUM_CODONS * _EMBED_DIM,), jnp.float32),
            pltpu.VMEM((_BW,), jnp.int32),
            pltpu.VMEM((_BW // 128, 8, 128), jnp.float32),
            pltpu.VMEM((_BW // 128, 8, 128), jnp.float32),
            pltpu.SemaphoreType.DMA,
            pltpu.SemaphoreType.DMA,
        ],
        compiler_params=pltpu.CompilerParams(
            use_tc_tiling_on_sc=True, needs_layout_passes=False),
    )(_gather_kernel)
    return kern(xt_flat, tableT_flat)


def kernel(x, embeddings):
    xt_flat = x.T.reshape(_N)
    tableT_flat = embeddings.T.reshape(_NUM_CODONS * _EMBED_DIM)
    out5 = _run(xt_flat, tableT_flat)
    return out5.transpose(2, 4, 0, 1, 3).reshape(_B, _T, _EMBED_DIM)


# speedup vs baseline: 8.4566x; 1.0204x over previous
import functools

import jax
import jax.numpy as jnp
from jax import lax
from jax.experimental import pallas as pl
from jax.experimental.pallas import tpu as pltpu
from jax.experimental.pallas import tpu_sc as plsc

_NUM_CODONS = 64
_EMBED_DIM = 16

_B = 16384
_T = 200
_N = _B * _T

_INFO = plsc.get_sparse_core_info()
_NC = _INFO.num_cores
_NS = _INFO.num_subcores
_NW = _NC * _NS
_BW = _B // _NW
_GRP = 16
_NG = _BW // _GRP


def _gather_kernel(xt_hbm, tableT_hbm, out_hbm, table_v, idx0_v, idx1_v,
                   buf00_v, buf01_v, buf10_v, buf11_v,
                   isem0, isem1, sem00, sem01, sem10, sem11):
    wid = lax.axis_index("s") * _NC + lax.axis_index("c")
    b0 = wid * _BW
    btile0 = wid * (_BW // 128)
    pltpu.sync_copy(tableT_hbm, table_v)
    idxs = (idx0_v, idx1_v)
    isems = (isem0, isem1)
    bufs = ((buf00_v, buf01_v), (buf10_v, buf11_v))
    sems = ((sem00, sem01), (sem10, sem11))

    def idx_copy(t, slot):
        return pltpu.make_async_copy(
            xt_hbm.at[pl.ds(t * _B + b0, _BW)], idxs[slot], isems[slot])

    def compute(idx_v, buf_v, dband):
        for bg in range(_NG):
            tok = idx_v[pl.ds(bg * _GRP, _GRP)]
            vals = [plsc.load_gather(table_v, [tok + (dband * 8 + ds) * _NUM_CODONS])
                    for ds in range(8)]
            for ds in range(8):
                buf_v[bg // 8, ds, pl.ds((bg % 8) * _GRP, _GRP)] = vals[ds]

    idx_copy(0, 0).start()

    def body(t, carry):
        par = lax.rem(t, 2)

        @pl.when(t + 1 < _T)
        def _prefetch():
            for slot in range(2):
                @pl.when(par != slot)
                def _go():
                    idx_copy(t + 1, slot).start()

        for slot in range(2):
            @pl.when(par == slot)
            def _step():
                idx_copy(t, slot).wait()
                for dband in range(2):
                    out_slice = out_hbm.at[t, dband, pl.ds(btile0, _BW // 128)]

                    @pl.when(t > 1)
                    def _drain():
                        pltpu.make_async_copy(
                            bufs[slot][dband], out_slice,
                            sems[slot][dband]).wait()

                    compute(idxs[slot], bufs[slot][dband], dband)
                    pltpu.async_copy(bufs[slot][dband], out_slice,
                                     sems[slot][dband])
        return carry

    lax.fori_loop(0, _T, body, 0)
    for slot in range(2):
        for dband in range(2):
            pltpu.make_async_copy(
                bufs[slot][dband],
                out_hbm.at[_T - 2 + slot, dband, pl.ds(btile0, _BW // 128)],
                sems[slot][dband]).wait()


@jax.jit
def _run(xt_flat, tableT_flat):
    mesh = plsc.VectorSubcoreMesh(core_axis_name="c", subcore_axis_name="s")
    kern = functools.partial(
        pl.kernel,
        mesh=mesh,
        out_type=jax.ShapeDtypeStruct((_T, 2, _B // 128, 8, 128), jnp.float32),
        scratch_types=[
            pltpu.VMEM((_NUM_CODONS * _EMBED_DIM,), jnp.float32),
            pltpu.VMEM((_BW,), jnp.int32),
            pltpu.VMEM((_BW,), jnp.int32),
            pltpu.VMEM((_BW // 128, 8, 128), jnp.float32),
            pltpu.VMEM((_BW // 128, 8, 128), jnp.float32),
            pltpu.VMEM((_BW // 128, 8, 128), jnp.float32),
            pltpu.VMEM((_BW // 128, 8, 128), jnp.float32),
            pltpu.SemaphoreType.DMA,
            pltpu.SemaphoreType.DMA,
            pltpu.SemaphoreType.DMA,
            pltpu.SemaphoreType.DMA,
            pltpu.SemaphoreType.DMA,
            pltpu.SemaphoreType.DMA,
        ],
        compiler_params=pltpu.CompilerParams(
            use_tc_tiling_on_sc=True, needs_layout_passes=False),
    )(_gather_kernel)
    return kern(xt_flat, tableT_flat)


def kernel(x, embeddings):
    xt_flat = x.T.reshape(_N)
    tableT_flat = embeddings.T.reshape(_NUM_CODONS * _EMBED_DIM)
    out5 = _run(xt_flat, tableT_flat)
    return out5.transpose(2, 4, 0, 1, 3).reshape(_B, _T, _EMBED_DIM)
